# baseline (speedup 1.0000x reference)
import functools

import jax
import jax.numpy as jnp
from jax import lax
from jax.experimental import pallas as pl
from jax.experimental.pallas import tpu as pltpu
from jax.experimental.pallas import tpu_sc as plsc

E = 8
KTOP = 2
RB = 256
CHUNK = 256
NW = 32
GCH = 64


def _router_plan_kernel(hs_ref, gw_ref,
                        logits_ref, w01_ref, pos0_ref, pos1_ref, be_ref,
                        hs16_ref, r0_ref, oh0_ref, r1_ref, oh1_ref):
    T = hs_ref.shape[0]
    nch = T // CHUNK
    nb = be_ref.shape[1]

    tri = (lax.broadcasted_iota(jnp.int32, (CHUNK, CHUNK), 0)
           > lax.broadcasted_iota(jnp.int32, (CHUNK, CHUNK), 1)
           ).astype(jnp.float32)
    eio = lax.broadcasted_iota(jnp.int32, (CHUNK, E), 1)

    def pass1(c, carry):
        c0, c1 = carry
        sl = pl.ds(c * CHUNK, CHUNK)
        x = hs_ref[sl, :]
        hs16_ref[sl, :] = x.astype(jnp.bfloat16)
        logits = lax.dot_general(x, gw_ref[...], (((1,), (1,)), ((), ())),
                                 preferred_element_type=jnp.float32)
        logits_ref[sl, :] = logits
        m = jnp.max(logits, axis=1, keepdims=True)
        p = jnp.exp(logits - m)
        p = p / jnp.sum(p, axis=1, keepdims=True)

        top0 = jnp.max(p, axis=1, keepdims=True)
        e0 = jnp.min(jnp.where(p == top0, eio, E), axis=1, keepdims=True)
        oh0 = (eio == e0).astype(jnp.float32)
        pm = jnp.where(oh0 > 0, -1.0, p)
        top1 = jnp.max(pm, axis=1, keepdims=True)
        e1 = jnp.min(jnp.where(pm == top1, eio, E), axis=1, keepdims=True)
        oh1 = (eio == e1).astype(jnp.float32)
        s = top0 + top1
        w01_ref[sl, :] = jnp.concatenate([top0 / s, top1 / s], axis=1)

        cum0 = lax.dot_general(tri, oh0, (((1,), (0,)), ((), ())),
                               preferred_element_type=jnp.float32,
                               precision=lax.Precision.HIGHEST) + c0
        cum1 = lax.dot_general(tri, oh1, (((1,), (0,)), ((), ())),
                               preferred_element_type=jnp.float32,
                               precision=lax.Precision.HIGHEST) + c1
        r0_ref[sl, :] = cum0 * oh0
        oh0_ref[sl, :] = oh0
        r1_ref[sl, :] = cum1 * oh1
        oh1_ref[sl, :] = oh1
        c0 = c0 + jnp.sum(oh0, axis=0, keepdims=True)
        c1 = c1 + jnp.sum(oh1, axis=0, keepdims=True)
        return (c0, c1)

    z = jnp.zeros((1, E), jnp.float32)
    c0, c1 = lax.fori_loop(0, nch, pass1, (z, z))

    counts = c0 + c1
    pc = jnp.ceil(counts / RB) * RB
    trie = (lax.broadcasted_iota(jnp.int32, (E, E), 0)
            < lax.broadcasted_iota(jnp.int32, (E, E), 1)).astype(jnp.float32)
    off = lax.dot_general(pc, trie, (((1,), (0,)), ((), ())),
                          preferred_element_type=jnp.float32,
                          precision=lax.Precision.HIGHEST)

    jio = lax.broadcasted_iota(jnp.int32, (1, nb), 1).astype(jnp.float32)
    be = jnp.zeros((1, nb), jnp.float32)
    for e in range(E):
        start = off[0, e] / RB
        nblk = pc[0, e] / RB
        be = be + e * jnp.where((jio >= start) & (jio < start + nblk), 1.0, 0.0)
    be_ref[...] = be.astype(jnp.int32)

    def pass2(c, _):
        sl = pl.ds(c * CHUNK, CHUNK)
        pos0 = jnp.sum(oh0_ref[sl, :] * off + r0_ref[sl, :],
                       axis=1, keepdims=True)
        pos1 = jnp.sum(oh1_ref[sl, :] * (off + c0) + r1_ref[sl, :],
                       axis=1, keepdims=True)
        pos0_ref[sl, :] = pos0.astype(jnp.int32)
        pos1_ref[sl, :] = pos1.astype(jnp.int32)
        return 0

    lax.fori_loop(0, nch, pass2, 0)


def _router_plan(hs, gate_w, nb):
    T = hs.shape[0]
    out_shapes = (
        jax.ShapeDtypeStruct((T, E), jnp.float32),
        jax.ShapeDtypeStruct((T, KTOP), jnp.float32),
        jax.ShapeDtypeStruct((T, 1), jnp.int32),
        jax.ShapeDtypeStruct((T, 1), jnp.int32),
        jax.ShapeDtypeStruct((1, nb), jnp.int32),
        jax.ShapeDtypeStruct((T, hs.shape[1]), jnp.bfloat16),
    )
    return pl.pallas_call(
        _router_plan_kernel,
        out_shape=out_shapes,
        scratch_shapes=[pltpu.VMEM((T, E), jnp.float32)] * 4,
    )(hs, gate_w)


def _dispatch_body(hs_hbm, pos_hbm, out_hbm, idx_v, rows_v, sem):
    n = pos_hbm.shape[0]
    t = hs_hbm.shape[0]
    per_w = n // NW
    wid = lax.axis_index("s") * 2 + lax.axis_index("c")
    base = wid * per_w

    def chunk(c, _):
        b = base + c * GCH
        srow = b - jnp.where(b >= t, t, 0)
        pltpu.sync_copy(pos_hbm.at[pl.ds(b, GCH)], idx_v)
        pltpu.sync_copy(hs_hbm.at[pl.ds(srow, GCH)], rows_v)
        pltpu.async_copy(rows_v, out_hbm.at[idx_v], sem).wait()
        return 0

    lax.fori_loop(0, per_w // GCH, chunk, 0)


def _sc_dispatch(hs, pos_flat, p_rows):
    d = hs.shape[1]
    k = pl.kernel(
        _dispatch_body,
        out_type=jax.ShapeDtypeStruct((p_rows, d), hs.dtype),
        mesh=plsc.VectorSubcoreMesh(core_axis_name="c", subcore_axis_name="s"),
        scratch_types=[
            pltpu.VMEM((GCH,), jnp.int32),
            pltpu.VMEM((GCH, d), hs.dtype),
            pltpu.SemaphoreType.DMA,
        ],
    )
    return k(hs, pos_flat)


def _gather_body(table_hbm, idx_hbm, out_hbm, idx_v, rows_v, sem):
    n = out_hbm.shape[0]
    rows_per_w = n // NW
    wid = lax.axis_index("s") * 2 + lax.axis_index("c")
    base = wid * rows_per_w

    def chunk(c, _):
        b = base + c * GCH
        pltpu.sync_copy(idx_hbm.at[pl.ds(b, GCH)], idx_v)
        pltpu.async_copy(table_hbm.at[idx_v], rows_v, sem).wait()
        pltpu.sync_copy(rows_v, out_hbm.at[pl.ds(b, GCH)])
        return 0

    lax.fori_loop(0, rows_per_w // GCH, chunk, 0)


def _sc_gather_rows(table, idx, n_rows):
    d = table.shape[1]
    k = pl.kernel(
        _gather_body,
        out_type=jax.ShapeDtypeStruct((n_rows, d), table.dtype),
        mesh=plsc.VectorSubcoreMesh(core_axis_name="c", subcore_axis_name="s"),
        scratch_types=[
            pltpu.VMEM((GCH,), jnp.int32),
            pltpu.VMEM((GCH, d), table.dtype),
            pltpu.SemaphoreType.DMA,
        ],
    )
    return k(table, idx)


def _expert_mlp_kernel(be_ref, x_ref, gw_ref, uw_ref, dw_ref, y_ref):
    x = x_ref[...].astype(jnp.bfloat16)
    g = lax.dot_general(x, gw_ref[0], (((1,), (1,)), ((), ())),
                        preferred_element_type=jnp.float32)
    u = lax.dot_general(x, uw_ref[0], (((1,), (1,)), ((), ())),
                        preferred_element_type=jnp.float32)
    h = (g * lax.logistic(g) * u).astype(jnp.bfloat16)
    y_ref[...] = lax.dot_general(h, dw_ref[0], (((1,), (1,)), ((), ())),
                                 preferred_element_type=jnp.float32
                                 ).astype(jnp.bfloat16)


def _expert_mlp(be, x_sorted, egw, euw, edw):
    p_rows, d = x_sorted.shape
    ff = egw.shape[1]
    nb = p_rows // RB
    grid_spec = pltpu.PrefetchScalarGridSpec(
        num_scalar_prefetch=1,
        grid=(nb,),
        in_specs=[
            pl.BlockSpec((RB, d), lambda i, be: (i, 0)),
            pl.BlockSpec((1, ff, d), lambda i, be: (be[i], 0, 0)),
            pl.BlockSpec((1, ff, d), lambda i, be: (be[i], 0, 0)),
            pl.BlockSpec((1, d, ff), lambda i, be: (be[i], 0, 0)),
        ],
        out_specs=pl.BlockSpec((RB, d), lambda i, be: (i, 0)),
    )
    return pl.pallas_call(
        _expert_mlp_kernel,
        grid_spec=grid_spec,
        out_shape=jax.ShapeDtypeStruct((p_rows, d), jnp.bfloat16),
        compiler_params=pltpu.CompilerParams(vmem_limit_bytes=128 * 1024 * 1024),
    )(be, x_sorted, egw, euw, edw)


def _shared_combine_kernel(hs_ref, y0_ref, y1_ref, w_ref,
                           sg_ref, su_ref, sd_ref, seg_ref, out_ref):
    x = hs_ref[...]
    x16 = x.astype(jnp.bfloat16)
    g = lax.dot_general(x16, sg_ref[...], (((1,), (1,)), ((), ())),
                        preferred_element_type=jnp.float32)
    u = lax.dot_general(x16, su_ref[...], (((1,), (1,)), ((), ())),
                        preferred_element_type=jnp.float32)
    h = (g * lax.logistic(g) * u).astype(jnp.bfloat16)
    s = lax.dot_general(h, sd_ref[...], (((1,), (1,)), ((), ())),
                        preferred_element_type=jnp.float32)
    gate = lax.logistic(lax.dot_general(x, seg_ref[...],
                                        (((1,), (1,)), ((), ())),
                                        preferred_element_type=jnp.float32))
    w = w_ref[...]
    out_ref[...] = (w[:, 0:1] * y0_ref[...].astype(jnp.float32)
                    + w[:, 1:2] * y1_ref[...].astype(jnp.float32)
                    + gate * s)


def _shared_combine(hs, yg, w01, sgw, suw, sdw, segw):
    T, d = hs.shape
    ff = sgw.shape[0]
    nch = T // CHUNK
    return pl.pallas_call(
        _shared_combine_kernel,
        grid=(nch,),
        in_specs=[
            pl.BlockSpec((CHUNK, d), lambda i: (i, 0)),
            pl.BlockSpec((CHUNK, d), lambda i: (i, 0)),
            pl.BlockSpec((CHUNK, d), lambda i: (i + nch, 0)),
            pl.BlockSpec((CHUNK, KTOP), lambda i: (i, 0)),
            pl.BlockSpec((ff, d), lambda i: (0, 0)),
            pl.BlockSpec((ff, d), lambda i: (0, 0)),
            pl.BlockSpec((d, ff), lambda i: (0, 0)),
            pl.BlockSpec((1, d), lambda i: (0, 0)),
        ],
        out_specs=pl.BlockSpec((CHUNK, d), lambda i: (i, 0)),
        out_shape=jax.ShapeDtypeStruct((T, d), jnp.float32),
        compiler_params=pltpu.CompilerParams(vmem_limit_bytes=128 * 1024 * 1024),
    )(hs, yg, yg, w01, sgw, suw, sdw, segw)


def kernel(hidden_states, gate_w, expert_gate_w, expert_up_w, expert_down_w,
           shared_gate_w, shared_up_w, shared_down_w, shared_expert_gate_w):
    B, S, Dm = hidden_states.shape
    hs = hidden_states.reshape(-1, Dm)
    T = hs.shape[0]
    nb = (KTOP * T + E * RB) // RB
    p_rows = nb * RB

    logits, w01, pos0, pos1, be, hs16 = _router_plan(hs, gate_w, nb)

    pos_flat = jnp.concatenate([pos0[:, 0], pos1[:, 0]])

    hs_i32 = lax.bitcast_convert_type(hs16.reshape(T, Dm // 2, 2), jnp.int32)
    xs_i32 = _sc_dispatch(hs_i32, pos_flat, p_rows)
    x_sorted = lax.bitcast_convert_type(
        xs_i32, jnp.bfloat16).reshape(p_rows, Dm)
    y_sorted = _expert_mlp(be.reshape(nb), x_sorted,
                           expert_gate_w.astype(jnp.bfloat16),
                           expert_up_w.astype(jnp.bfloat16),
                           expert_down_w.astype(jnp.bfloat16))
    ys_i32 = lax.bitcast_convert_type(
        y_sorted.reshape(p_rows, Dm // 2, 2), jnp.int32)
    yg_i32 = _sc_gather_rows(ys_i32, pos_flat, KTOP * T)
    yg = lax.bitcast_convert_type(
        yg_i32, jnp.bfloat16).reshape(KTOP * T, Dm)
    final = _shared_combine(hs, yg, w01,
                            shared_gate_w.astype(jnp.bfloat16),
                            shared_up_w.astype(jnp.bfloat16),
                            shared_down_w.astype(jnp.bfloat16),
                            shared_expert_gate_w)
    return final.reshape(B, S, Dm), logits


# speedup vs baseline: 1.8073x; 1.8073x over previous
import functools

import jax
import jax.numpy as jnp
from jax import lax
from jax.experimental import pallas as pl
from jax.experimental.pallas import tpu as pltpu
from jax.experimental.pallas import tpu_sc as plsc

E = 8
KTOP = 2
RB = 256
CHUNK = 256
NW = 32
GCH = 64


def _router_plan_kernel(hs_ref, gw_ref, seg_ref,
                        logits_ref, w01_ref, pos0_ref, pos1_ref, be_ref,
                        sgate_ref, r0_ref, oh0_ref, r1_ref, oh1_ref):
    T = hs_ref.shape[0]
    nch = T // CHUNK
    nb = be_ref.shape[1]

    tri = (lax.broadcasted_iota(jnp.int32, (CHUNK, CHUNK), 0)
           > lax.broadcasted_iota(jnp.int32, (CHUNK, CHUNK), 1)
           ).astype(jnp.float32)
    eio = lax.broadcasted_iota(jnp.int32, (CHUNK, E), 1)

    def pass1(c, carry):
        c0, c1 = carry
        sl = pl.ds(c * CHUNK, CHUNK)
        x = hs_ref[sl, :]
        logits = lax.dot_general(x, gw_ref[...], (((1,), (1,)), ((), ())),
                                 preferred_element_type=jnp.float32)
        logits_ref[sl, :] = logits
        sgate_ref[sl, :] = lax.logistic(
            lax.dot_general(x, seg_ref[...], (((1,), (1,)), ((), ())),
                            preferred_element_type=jnp.float32))
        m = jnp.max(logits, axis=1, keepdims=True)
        p = jnp.exp(logits - m)
        p = p / jnp.sum(p, axis=1, keepdims=True)

        top0 = jnp.max(p, axis=1, keepdims=True)
        e0 = jnp.min(jnp.where(p == top0, eio, E), axis=1, keepdims=True)
        oh0 = (eio == e0).astype(jnp.float32)
        pm = jnp.where(oh0 > 0, -1.0, p)
        top1 = jnp.max(pm, axis=1, keepdims=True)
        e1 = jnp.min(jnp.where(pm == top1, eio, E), axis=1, keepdims=True)
        oh1 = (eio == e1).astype(jnp.float32)
        s = top0 + top1
        w01_ref[sl, :] = jnp.concatenate([top0 / s, top1 / s], axis=1)

        cum0 = lax.dot_general(tri, oh0, (((1,), (0,)), ((), ())),
                               preferred_element_type=jnp.float32,
                               precision=lax.Precision.HIGHEST) + c0
        cum1 = lax.dot_general(tri, oh1, (((1,), (0,)), ((), ())),
                               preferred_element_type=jnp.float32,
                               precision=lax.Precision.HIGHEST) + c1
        r0_ref[sl, :] = cum0 * oh0
        oh0_ref[sl, :] = oh0
        r1_ref[sl, :] = cum1 * oh1
        oh1_ref[sl, :] = oh1
        c0 = c0 + jnp.sum(oh0, axis=0, keepdims=True)
        c1 = c1 + jnp.sum(oh1, axis=0, keepdims=True)
        return (c0, c1)

    z = jnp.zeros((1, E), jnp.float32)
    c0, c1 = lax.fori_loop(0, nch, pass1, (z, z))

    counts = c0 + c1
    pc = jnp.ceil(counts / RB) * RB
    trie = (lax.broadcasted_iota(jnp.int32, (E, E), 0)
            < lax.broadcasted_iota(jnp.int32, (E, E), 1)).astype(jnp.float32)
    off = lax.dot_general(pc, trie, (((1,), (0,)), ((), ())),
                          preferred_element_type=jnp.float32,
                          precision=lax.Precision.HIGHEST)

    jio = lax.broadcasted_iota(jnp.int32, (1, nb), 1).astype(jnp.float32)
    be = jnp.zeros((1, nb), jnp.float32)
    for e in range(E):
        start = off[0, e] / RB
        nblk = pc[0, e] / RB
        be = be + e * jnp.where((jio >= start) & (jio < start + nblk), 1.0, 0.0)
    be_ref[...] = be.astype(jnp.int32)

    def pass2(c, _):
        sl = pl.ds(c * CHUNK, CHUNK)
        pos0 = jnp.sum(oh0_ref[sl, :] * off + r0_ref[sl, :],
                       axis=1, keepdims=True)
        pos1 = jnp.sum(oh1_ref[sl, :] * (off + c0) + r1_ref[sl, :],
                       axis=1, keepdims=True)
        pos0_ref[sl, :] = pos0.astype(jnp.int32)
        pos1_ref[sl, :] = pos1.astype(jnp.int32)
        return 0

    lax.fori_loop(0, nch, pass2, 0)


def _router_plan(hs, gate_w, segw, nb):
    T = hs.shape[0]
    out_shapes = (
        jax.ShapeDtypeStruct((T, E), jnp.float32),
        jax.ShapeDtypeStruct((T, KTOP), jnp.float32),
        jax.ShapeDtypeStruct((T, 1), jnp.int32),
        jax.ShapeDtypeStruct((T, 1), jnp.int32),
        jax.ShapeDtypeStruct((1, nb), jnp.int32),
        jax.ShapeDtypeStruct((T, 1), jnp.float32),
    )
    return pl.pallas_call(
        _router_plan_kernel,
        out_shape=out_shapes,
        scratch_shapes=[pltpu.VMEM((T, E), jnp.float32)] * 4,
    )(hs, gate_w, segw)


def _dispatch_body(hs_hbm, pos_hbm, out_hbm, idx_v, rows_v, sem):
    n = pos_hbm.shape[0]
    t = hs_hbm.shape[0]
    per_w = n // NW
    wid = lax.axis_index("s") * 2 + lax.axis_index("c")
    base = wid * per_w

    def chunk(c, _):
        b = base + c * GCH
        srow = b - jnp.where(b >= t, t, 0)
        pltpu.sync_copy(pos_hbm.at[pl.ds(b, GCH)], idx_v)
        pltpu.sync_copy(hs_hbm.at[pl.ds(srow, GCH)], rows_v)
        pltpu.async_copy(rows_v, out_hbm.at[idx_v], sem).wait()
        return 0

    lax.fori_loop(0, per_w // GCH, chunk, 0)


def _sc_dispatch(hs, pos_flat, p_rows):
    d = hs.shape[1]
    k = pl.kernel(
        _dispatch_body,
        out_type=jax.ShapeDtypeStruct((p_rows, d), hs.dtype),
        mesh=plsc.VectorSubcoreMesh(core_axis_name="c", subcore_axis_name="s"),
        scratch_types=[
            pltpu.VMEM((GCH,), jnp.int32),
            pltpu.VMEM((GCH, d), hs.dtype),
            pltpu.SemaphoreType.DMA,
        ],
    )
    return k(hs, pos_flat)


def _gather_body(table_hbm, idx_hbm, out_hbm, idx_v, rows_v, sem):
    n = out_hbm.shape[0]
    rows_per_w = n // NW
    wid = lax.axis_index("s") * 2 + lax.axis_index("c")
    base = wid * rows_per_w

    def chunk(c, _):
        b = base + c * GCH
        pltpu.sync_copy(idx_hbm.at[pl.ds(b, GCH)], idx_v)
        pltpu.async_copy(table_hbm.at[idx_v], rows_v, sem).wait()
        pltpu.sync_copy(rows_v, out_hbm.at[pl.ds(b, GCH)])
        return 0

    lax.fori_loop(0, rows_per_w // GCH, chunk, 0)


def _sc_gather_rows(table, idx, n_rows):
    d = table.shape[1]
    k = pl.kernel(
        _gather_body,
        out_type=jax.ShapeDtypeStruct((n_rows, d), table.dtype),
        mesh=plsc.VectorSubcoreMesh(core_axis_name="c", subcore_axis_name="s"),
        scratch_types=[
            pltpu.VMEM((GCH,), jnp.int32),
            pltpu.VMEM((GCH, d), table.dtype),
            pltpu.SemaphoreType.DMA,
        ],
    )
    return k(table, idx)


def _expert_mlp_kernel(nb, be_ref, xs_ref, hsx_ref, gw_ref, uw_ref, dw_ref,
                       y_ref):
    pid = pl.program_id(0)
    x = jnp.where(pid < nb, xs_ref[...], hsx_ref[...])
    x = x.astype(jnp.bfloat16)
    g = lax.dot_general(x, gw_ref[0], (((1,), (1,)), ((), ())),
                        preferred_element_type=jnp.float32)
    u = lax.dot_general(x, uw_ref[0], (((1,), (1,)), ((), ())),
                        preferred_element_type=jnp.float32)
    h = (g * lax.logistic(g) * u).astype(jnp.bfloat16)
    y_ref[...] = lax.dot_general(h, dw_ref[0], (((1,), (1,)), ((), ())),
                                 preferred_element_type=jnp.float32)


def _expert_mlp(be2, x_sorted, hs, egw9, euw9, edw9):
    p_rows, d = x_sorted.shape
    t = hs.shape[0]
    ff = egw9.shape[1]
    nb = p_rows // RB
    nsh = t // RB
    grid_spec = pltpu.PrefetchScalarGridSpec(
        num_scalar_prefetch=1,
        grid=(nb + nsh,),
        in_specs=[
            pl.BlockSpec((RB, d), lambda i, be: (jnp.minimum(i, nb - 1), 0)),
            pl.BlockSpec((RB, d),
                         lambda i, be: (jnp.maximum(i - nb, 0), 0)),
            pl.BlockSpec((1, ff, d), lambda i, be: (be[i], 0, 0)),
            pl.BlockSpec((1, ff, d), lambda i, be: (be[i], 0, 0)),
            pl.BlockSpec((1, d, ff), lambda i, be: (be[i], 0, 0)),
        ],
        out_specs=pl.BlockSpec((RB, d), lambda i, be: (i, 0)),
    )
    return pl.pallas_call(
        functools.partial(_expert_mlp_kernel, nb),
        grid_spec=grid_spec,
        out_shape=jax.ShapeDtypeStruct((p_rows + t, d), jnp.float32),
        compiler_params=pltpu.CompilerParams(vmem_limit_bytes=128 * 1024 * 1024),
    )(be2, x_sorted, hs, egw9, euw9, edw9)


def _combine_kernel(y0_ref, y1_ref, ysh_ref, w_ref, sg_ref, out_ref):
    w = w_ref[...]
    out_ref[...] = (w[:, 0:1] * y0_ref[...] + w[:, 1:2] * y1_ref[...]
                    + sg_ref[...] * ysh_ref[...])


def _combine(yg, y_cat, w01, sgate, p_rows):
    T, d = sgate.shape[0], yg.shape[1]
    nch = T // CHUNK
    nb = p_rows // CHUNK
    return pl.pallas_call(
        _combine_kernel,
        grid=(nch,),
        in_specs=[
            pl.BlockSpec((CHUNK, d), lambda i: (i, 0)),
            pl.BlockSpec((CHUNK, d), lambda i: (i + nch, 0)),
            pl.BlockSpec((CHUNK, d), lambda i: (i + nb, 0)),
            pl.BlockSpec((CHUNK, KTOP), lambda i: (i, 0)),
            pl.BlockSpec((CHUNK, 1), lambda i: (i, 0)),
        ],
        out_specs=pl.BlockSpec((CHUNK, d), lambda i: (i, 0)),
        out_shape=jax.ShapeDtypeStruct((T, d), jnp.float32),
    )(yg, yg, y_cat, w01, sgate)


def kernel(hidden_states, gate_w, expert_gate_w, expert_up_w, expert_down_w,
           shared_gate_w, shared_up_w, shared_down_w, shared_expert_gate_w):
    B, S, Dm = hidden_states.shape
    hs = hidden_states.reshape(-1, Dm)
    T = hs.shape[0]
    nb = (KTOP * T + E * RB) // RB
    p_rows = nb * RB

    logits, w01, pos0, pos1, be, sgate = _router_plan(
        hs, gate_w, shared_expert_gate_w, nb)

    pos_flat = jnp.concatenate([pos0[:, 0], pos1[:, 0]])

    egw9 = jnp.concatenate([expert_gate_w.astype(jnp.bfloat16),
                            shared_gate_w.astype(jnp.bfloat16)[None]])
    euw9 = jnp.concatenate([expert_up_w.astype(jnp.bfloat16),
                            shared_up_w.astype(jnp.bfloat16)[None]])
    edw9 = jnp.concatenate([expert_down_w.astype(jnp.bfloat16),
                            shared_down_w.astype(jnp.bfloat16)[None]])
    be2 = jnp.concatenate([be.reshape(nb),
                           jnp.full((T // RB,), E, dtype=jnp.int32)])

    x_sorted = _sc_dispatch(hs, pos_flat, p_rows)
    y_cat = _expert_mlp(be2, x_sorted, hs, egw9, euw9, edw9)
    yg = _sc_gather_rows(y_cat, pos_flat, KTOP * T)
    final = _combine(yg, y_cat, w01, sgate, p_rows)
    return final.reshape(B, S, Dm), logits


# speedup vs baseline: 2.1106x; 1.1679x over previous
import functools

import jax
import jax.numpy as jnp
from jax import lax
from jax.experimental import pallas as pl
from jax.experimental.pallas import tpu as pltpu
from jax.experimental.pallas import tpu_sc as plsc

E = 8
KTOP = 2
RB = 256
CHUNK = 256
NW = 32
GCH = 64


def _router_plan_kernel(hs_ref, gw_ref, seg_ref,
                        logits_ref, w01_ref, pos0_ref, pos1_ref, be_ref,
                        sgate_ref, r0_ref, oh0_ref, r1_ref, oh1_ref):
    T = hs_ref.shape[0]
    nch = T // CHUNK
    nb = be_ref.shape[1]

    tri = (lax.broadcasted_iota(jnp.int32, (CHUNK, CHUNK), 0)
           > lax.broadcasted_iota(jnp.int32, (CHUNK, CHUNK), 1)
           ).astype(jnp.float32)
    eio = lax.broadcasted_iota(jnp.int32, (CHUNK, E), 1)

    def pass1(c, carry):
        c0, c1 = carry
        sl = pl.ds(c * CHUNK, CHUNK)
        x = hs_ref[sl, :]
        logits = lax.dot_general(x, gw_ref[...], (((1,), (1,)), ((), ())),
                                 preferred_element_type=jnp.float32)
        logits_ref[sl, :] = logits
        sgate_ref[sl, :] = lax.logistic(
            lax.dot_general(x, seg_ref[...], (((1,), (1,)), ((), ())),
                            preferred_element_type=jnp.float32))
        m = jnp.max(logits, axis=1, keepdims=True)
        p = jnp.exp(logits - m)
        p = p / jnp.sum(p, axis=1, keepdims=True)

        top0 = jnp.max(p, axis=1, keepdims=True)
        e0 = jnp.min(jnp.where(p == top0, eio, E), axis=1, keepdims=True)
        oh0 = (eio == e0).astype(jnp.float32)
        pm = jnp.where(oh0 > 0, -1.0, p)
        top1 = jnp.max(pm, axis=1, keepdims=True)
        e1 = jnp.min(jnp.where(pm == top1, eio, E), axis=1, keepdims=True)
        oh1 = (eio == e1).astype(jnp.float32)
        s = top0 + top1
        w01_ref[sl, :] = jnp.concatenate([top0 / s, top1 / s], axis=1)

        cum0 = lax.dot_general(tri, oh0, (((1,), (0,)), ((), ())),
                               preferred_element_type=jnp.float32,
                               precision=lax.Precision.HIGHEST) + c0
        cum1 = lax.dot_general(tri, oh1, (((1,), (0,)), ((), ())),
                               preferred_element_type=jnp.float32,
                               precision=lax.Precision.HIGHEST) + c1
        r0_ref[sl, :] = cum0 * oh0
        oh0_ref[sl, :] = oh0
        r1_ref[sl, :] = cum1 * oh1
        oh1_ref[sl, :] = oh1
        c0 = c0 + jnp.sum(oh0, axis=0, keepdims=True)
        c1 = c1 + jnp.sum(oh1, axis=0, keepdims=True)
        return (c0, c1)

    z = jnp.zeros((1, E), jnp.float32)
    c0, c1 = lax.fori_loop(0, nch, pass1, (z, z))

    counts = c0 + c1
    pc = jnp.ceil(counts / RB) * RB
    trie = (lax.broadcasted_iota(jnp.int32, (E, E), 0)
            < lax.broadcasted_iota(jnp.int32, (E, E), 1)).astype(jnp.float32)
    off = lax.dot_general(pc, trie, (((1,), (0,)), ((), ())),
                          preferred_element_type=jnp.float32,
                          precision=lax.Precision.HIGHEST)

    jio = lax.broadcasted_iota(jnp.int32, (1, nb), 1).astype(jnp.float32)
    be = jnp.zeros((1, nb), jnp.float32)
    for e in range(E):
        start = off[0, e] / RB
        nblk = pc[0, e] / RB
        be = be + e * jnp.where((jio >= start) & (jio < start + nblk), 1.0, 0.0)
    be_ref[...] = be.astype(jnp.int32)

    def pass2(c, _):
        sl = pl.ds(c * CHUNK, CHUNK)
        pos0 = jnp.sum(oh0_ref[sl, :] * off + r0_ref[sl, :],
                       axis=1, keepdims=True)
        pos1 = jnp.sum(oh1_ref[sl, :] * (off + c0) + r1_ref[sl, :],
                       axis=1, keepdims=True)
        pos0_ref[sl, :] = pos0.astype(jnp.int32)
        pos1_ref[sl, :] = pos1.astype(jnp.int32)
        return 0

    lax.fori_loop(0, nch, pass2, 0)


def _router_plan(hs, gate_w, segw, nb):
    T = hs.shape[0]
    out_shapes = (
        jax.ShapeDtypeStruct((T, E), jnp.float32),
        jax.ShapeDtypeStruct((T, KTOP), jnp.float32),
        jax.ShapeDtypeStruct((T, 1), jnp.int32),
        jax.ShapeDtypeStruct((T, 1), jnp.int32),
        jax.ShapeDtypeStruct((1, nb), jnp.int32),
        jax.ShapeDtypeStruct((T, 1), jnp.float32),
    )
    return pl.pallas_call(
        _router_plan_kernel,
        out_shape=out_shapes,
        scratch_shapes=[pltpu.VMEM((T, E), jnp.float32)] * 4,
    )(hs, gate_w, segw)


def _dispatch_body(hs_hbm, pos_hbm, out_hbm, idx_v, rows_v, sem):
    n = pos_hbm.shape[0]
    t = hs_hbm.shape[0]
    per_w = n // NW
    wid = lax.axis_index("s") * 2 + lax.axis_index("c")
    base = wid * per_w

    def chunk(c, _):
        b = base + c * GCH
        srow = b - jnp.where(b >= t, t, 0)
        pltpu.sync_copy(pos_hbm.at[pl.ds(b, GCH)], idx_v)
        pltpu.sync_copy(hs_hbm.at[pl.ds(srow, GCH)], rows_v)
        pltpu.async_copy(rows_v, out_hbm.at[idx_v], sem).wait()
        return 0

    lax.fori_loop(0, per_w // GCH, chunk, 0)


def _sc_dispatch(hs, pos_flat, p_rows):
    d = hs.shape[1]
    k = pl.kernel(
        _dispatch_body,
        out_type=jax.ShapeDtypeStruct((p_rows, d), hs.dtype),
        mesh=plsc.VectorSubcoreMesh(core_axis_name="c", subcore_axis_name="s"),
        scratch_types=[
            pltpu.VMEM((GCH,), jnp.int32),
            pltpu.VMEM((GCH, d), hs.dtype),
            pltpu.SemaphoreType.DMA,
        ],
    )
    return k(hs, pos_flat)


def _gather_body(table_hbm, idx_hbm, out_hbm, idx_v, rows_v, sem):
    n = out_hbm.shape[0]
    rows_per_w = n // NW
    wid = lax.axis_index("s") * 2 + lax.axis_index("c")
    base = wid * rows_per_w

    def chunk(c, _):
        b = base + c * GCH
        pltpu.sync_copy(idx_hbm.at[pl.ds(b, GCH)], idx_v)
        pltpu.async_copy(table_hbm.at[idx_v], rows_v, sem).wait()
        pltpu.sync_copy(rows_v, out_hbm.at[pl.ds(b, GCH)])
        return 0

    lax.fori_loop(0, rows_per_w // GCH, chunk, 0)


def _sc_gather_rows(table, idx, n_rows):
    d = table.shape[1]
    k = pl.kernel(
        _gather_body,
        out_type=jax.ShapeDtypeStruct((n_rows, d), table.dtype),
        mesh=plsc.VectorSubcoreMesh(core_axis_name="c", subcore_axis_name="s"),
        scratch_types=[
            pltpu.VMEM((GCH,), jnp.int32),
            pltpu.VMEM((GCH, d), table.dtype),
            pltpu.SemaphoreType.DMA,
        ],
    )
    return k(table, idx)


def _expert_mlp_kernel(be_ref, xs_ref, gw_ref, uw_ref, dw_ref, y_ref):
    x = xs_ref[...].astype(jnp.bfloat16)
    g = lax.dot_general(x, gw_ref[0], (((1,), (1,)), ((), ())),
                        preferred_element_type=jnp.float32)
    u = lax.dot_general(x, uw_ref[0], (((1,), (1,)), ((), ())),
                        preferred_element_type=jnp.float32)
    h = (g * lax.logistic(g) * u).astype(jnp.bfloat16)
    y_ref[...] = lax.dot_general(h, dw_ref[0], (((1,), (1,)), ((), ())),
                                 preferred_element_type=jnp.float32)


def _expert_mlp(be, x_sorted, egw, euw, edw):
    p_rows, d = x_sorted.shape
    ff = egw.shape[1]
    nb = p_rows // RB
    grid_spec = pltpu.PrefetchScalarGridSpec(
        num_scalar_prefetch=1,
        grid=(nb,),
        in_specs=[
            pl.BlockSpec((RB, d), lambda i, be: (i, 0)),
            pl.BlockSpec((1, ff, d), lambda i, be: (be[i], 0, 0)),
            pl.BlockSpec((1, ff, d), lambda i, be: (be[i], 0, 0)),
            pl.BlockSpec((1, d, ff), lambda i, be: (be[i], 0, 0)),
        ],
        out_specs=pl.BlockSpec((RB, d), lambda i, be: (i, 0)),
    )
    return pl.pallas_call(
        _expert_mlp_kernel,
        grid_spec=grid_spec,
        out_shape=jax.ShapeDtypeStruct((p_rows, d), jnp.float32),
        compiler_params=pltpu.CompilerParams(vmem_limit_bytes=128 * 1024 * 1024),
    )(be, x_sorted, egw, euw, edw)


def _shared_mlp_kernel(hs_ref, sg_ref, su_ref, sd_ref, sgate_ref, out_ref):
    x16 = hs_ref[...].astype(jnp.bfloat16)
    g = lax.dot_general(x16, sg_ref[...], (((1,), (1,)), ((), ())),
                        preferred_element_type=jnp.float32)
    u = lax.dot_general(x16, su_ref[...], (((1,), (1,)), ((), ())),
                        preferred_element_type=jnp.float32)
    h = (g * lax.logistic(g) * u).astype(jnp.bfloat16)
    s = lax.dot_general(h, sd_ref[...], (((1,), (1,)), ((), ())),
                        preferred_element_type=jnp.float32)
    out_ref[...] = sgate_ref[...] * s


def _shared_mlp(hs, sgw, suw, sdw, sgate):
    T, d = hs.shape
    ff = sgw.shape[0]
    nch = T // CHUNK
    return pl.pallas_call(
        _shared_mlp_kernel,
        grid=(nch,),
        in_specs=[
            pl.BlockSpec((CHUNK, d), lambda i: (i, 0)),
            pl.BlockSpec((ff, d), lambda i: (0, 0)),
            pl.BlockSpec((ff, d), lambda i: (0, 0)),
            pl.BlockSpec((d, ff), lambda i: (0, 0)),
            pl.BlockSpec((CHUNK, 1), lambda i: (i, 0)),
        ],
        out_specs=pl.BlockSpec((CHUNK, d), lambda i: (i, 0)),
        out_shape=jax.ShapeDtypeStruct((T, d), jnp.float32),
        compiler_params=pltpu.CompilerParams(vmem_limit_bytes=128 * 1024 * 1024),
    )(hs, sgw, suw, sdw, sgate)


def _combine_kernel(y0_ref, y1_ref, ysh_ref, w_ref, out_ref):
    w = w_ref[...]
    out_ref[...] = (w[:, 0:1] * y0_ref[...] + w[:, 1:2] * y1_ref[...]
                    + ysh_ref[...])


def _combine(yg, s_gated, w01):
    T, d = s_gated.shape
    nch = T // CHUNK
    return pl.pallas_call(
        _combine_kernel,
        grid=(nch,),
        in_specs=[
            pl.BlockSpec((CHUNK, d), lambda i: (i, 0)),
            pl.BlockSpec((CHUNK, d), lambda i: (i + nch, 0)),
            pl.BlockSpec((CHUNK, d), lambda i: (i, 0)),
            pl.BlockSpec((CHUNK, KTOP), lambda i: (i, 0)),
        ],
        out_specs=pl.BlockSpec((CHUNK, d), lambda i: (i, 0)),
        out_shape=jax.ShapeDtypeStruct((T, d), jnp.float32),
    )(yg, yg, s_gated, w01)


def kernel(hidden_states, gate_w, expert_gate_w, expert_up_w, expert_down_w,
           shared_gate_w, shared_up_w, shared_down_w, shared_expert_gate_w):
    B, S, Dm = hidden_states.shape
    hs = hidden_states.reshape(-1, Dm)
    T = hs.shape[0]
    nb = (KTOP * T + E * RB) // RB
    p_rows = nb * RB

    logits, w01, pos0, pos1, be, sgate = _router_plan(
        hs, gate_w, shared_expert_gate_w, nb)

    pos_flat = jnp.concatenate([pos0[:, 0], pos1[:, 0]])

    x_sorted = _sc_dispatch(hs, pos_flat, p_rows)
    y_sorted = _expert_mlp(be.reshape(nb), x_sorted,
                           expert_gate_w.astype(jnp.bfloat16),
                           expert_up_w.astype(jnp.bfloat16),
                           expert_down_w.astype(jnp.bfloat16))
    s_gated = _shared_mlp(hs, shared_gate_w.astype(jnp.bfloat16),
                          shared_up_w.astype(jnp.bfloat16),
                          shared_down_w.astype(jnp.bfloat16), sgate)
    yg = _sc_gather_rows(y_sorted, pos_flat, KTOP * T)
    final = _combine(yg, s_gated, w01)
    return final.reshape(B, S, Dm), logits


# speedup vs baseline: 2.2017x; 1.0431x over previous
import functools

import jax
import jax.numpy as jnp
from jax import lax
from jax.experimental import pallas as pl
from jax.experimental.pallas import tpu as pltpu
from jax.experimental.pallas import tpu_sc as plsc

E = 8
KTOP = 2
RB = 256
CHUNK = 256
NW = 32
GCH = 64


def _router_plan_kernel(hs_ref, gw_ref, seg_ref,
                        logits_ref, w01_ref, pos0_ref, pos1_ref, be_ref,
                        sgate_ref, r0_ref, oh0_ref, r1_ref, oh1_ref):
    T = hs_ref.shape[0]
    nch = T // CHUNK
    nb = be_ref.shape[1]

    tri = (lax.broadcasted_iota(jnp.int32, (CHUNK, CHUNK), 0)
           > lax.broadcasted_iota(jnp.int32, (CHUNK, CHUNK), 1)
           ).astype(jnp.float32)
    eio = lax.broadcasted_iota(jnp.int32, (CHUNK, E), 1)

    def pass1(c, carry):
        c0, c1 = carry
        sl = pl.ds(c * CHUNK, CHUNK)
        x = hs_ref[sl, :]
        logits = lax.dot_general(x, gw_ref[...], (((1,), (1,)), ((), ())),
                                 preferred_element_type=jnp.float32)
        logits_ref[sl, :] = logits
        sgate_ref[sl, :] = lax.logistic(
            lax.dot_general(x, seg_ref[...], (((1,), (1,)), ((), ())),
                            preferred_element_type=jnp.float32))
        m = jnp.max(logits, axis=1, keepdims=True)
        p = jnp.exp(logits - m)
        p = p / jnp.sum(p, axis=1, keepdims=True)

        top0 = jnp.max(p, axis=1, keepdims=True)
        e0 = jnp.min(jnp.where(p == top0, eio, E), axis=1, keepdims=True)
        oh0 = (eio == e0).astype(jnp.float32)
        pm = jnp.where(oh0 > 0, -1.0, p)
        top1 = jnp.max(pm, axis=1, keepdims=True)
        e1 = jnp.min(jnp.where(pm == top1, eio, E), axis=1, keepdims=True)
        oh1 = (eio == e1).astype(jnp.float32)
        s = top0 + top1
        w01_ref[sl, :] = jnp.concatenate([top0 / s, top1 / s], axis=1)

        cum0 = lax.dot_general(tri, oh0, (((1,), (0,)), ((), ())),
                               preferred_element_type=jnp.float32,
                               precision=lax.Precision.HIGHEST) + c0
        cum1 = lax.dot_general(tri, oh1, (((1,), (0,)), ((), ())),
                               preferred_element_type=jnp.float32,
                               precision=lax.Precision.HIGHEST) + c1
        r0_ref[sl, :] = cum0 * oh0
        oh0_ref[sl, :] = oh0
        r1_ref[sl, :] = cum1 * oh1
        oh1_ref[sl, :] = oh1
        c0 = c0 + jnp.sum(oh0, axis=0, keepdims=True)
        c1 = c1 + jnp.sum(oh1, axis=0, keepdims=True)
        return (c0, c1)

    z = jnp.zeros((1, E), jnp.float32)
    c0, c1 = lax.fori_loop(0, nch, pass1, (z, z))

    counts = c0 + c1
    pc = jnp.ceil(counts / RB) * RB
    trie = (lax.broadcasted_iota(jnp.int32, (E, E), 0)
            < lax.broadcasted_iota(jnp.int32, (E, E), 1)).astype(jnp.float32)
    off = lax.dot_general(pc, trie, (((1,), (0,)), ((), ())),
                          preferred_element_type=jnp.float32,
                          precision=lax.Precision.HIGHEST)

    jio = lax.broadcasted_iota(jnp.int32, (1, nb), 1).astype(jnp.float32)
    be = jnp.zeros((1, nb), jnp.float32)
    for e in range(E):
        start = off[0, e] / RB
        nblk = pc[0, e] / RB
        be = be + e * jnp.where((jio >= start) & (jio < start + nblk), 1.0, 0.0)
    be_ref[...] = be.astype(jnp.int32)

    def pass2(c, _):
        sl = pl.ds(c * CHUNK, CHUNK)
        pos0 = jnp.sum(oh0_ref[sl, :] * off + r0_ref[sl, :],
                       axis=1, keepdims=True)
        pos1 = jnp.sum(oh1_ref[sl, :] * (off + c0) + r1_ref[sl, :],
                       axis=1, keepdims=True)
        pos0_ref[sl, :] = pos0.astype(jnp.int32)
        pos1_ref[sl, :] = pos1.astype(jnp.int32)
        return 0

    lax.fori_loop(0, nch, pass2, 0)


def _router_plan(hs, gate_w, segw, nb):
    T = hs.shape[0]
    out_shapes = (
        jax.ShapeDtypeStruct((T, E), jnp.float32),
        jax.ShapeDtypeStruct((T, KTOP), jnp.float32),
        jax.ShapeDtypeStruct((T, 1), jnp.int32),
        jax.ShapeDtypeStruct((T, 1), jnp.int32),
        jax.ShapeDtypeStruct((1, nb), jnp.int32),
        jax.ShapeDtypeStruct((T, 1), jnp.float32),
    )
    return pl.pallas_call(
        _router_plan_kernel,
        out_shape=out_shapes,
        scratch_shapes=[pltpu.VMEM((T, E), jnp.float32)] * 4,
    )(hs, gate_w, segw)


def _dispatch_body(hs_hbm, pos_hbm, out_hbm, idx_v, rows_v, sem):
    n = pos_hbm.shape[0]
    t = hs_hbm.shape[0]
    per_w = n // NW
    wid = lax.axis_index("s") * 2 + lax.axis_index("c")
    base = wid * per_w

    def chunk(c, _):
        b = base + c * GCH
        srow = b - jnp.where(b >= t, t, 0)
        pltpu.sync_copy(pos_hbm.at[pl.ds(b, GCH)], idx_v)
        pltpu.sync_copy(hs_hbm.at[pl.ds(srow, GCH)], rows_v)
        pltpu.async_copy(rows_v, out_hbm.at[idx_v], sem).wait()
        return 0

    lax.fori_loop(0, per_w // GCH, chunk, 0)


def _sc_dispatch(hs, pos_flat, p_rows):
    d = hs.shape[1]
    k = pl.kernel(
        _dispatch_body,
        out_type=jax.ShapeDtypeStruct((p_rows, d), hs.dtype),
        mesh=plsc.VectorSubcoreMesh(core_axis_name="c", subcore_axis_name="s"),
        scratch_types=[
            pltpu.VMEM((GCH,), jnp.int32),
            pltpu.VMEM((GCH, d), hs.dtype),
            pltpu.SemaphoreType.DMA,
        ],
    )
    return k(hs, pos_flat)


def _gather_body(table_hbm, idx_hbm, out_hbm, idx_v, rows_v, sem):
    n = out_hbm.shape[0]
    rows_per_w = n // NW
    wid = lax.axis_index("s") * 2 + lax.axis_index("c")
    base = wid * rows_per_w

    def chunk(c, _):
        b = base + c * GCH
        pltpu.sync_copy(idx_hbm.at[pl.ds(b, GCH)], idx_v)
        pltpu.async_copy(table_hbm.at[idx_v], rows_v, sem).wait()
        pltpu.sync_copy(rows_v, out_hbm.at[pl.ds(b, GCH)])
        return 0

    lax.fori_loop(0, rows_per_w // GCH, chunk, 0)


def _sc_gather_rows(table, idx, n_rows):
    d = table.shape[1]
    k = pl.kernel(
        _gather_body,
        out_type=jax.ShapeDtypeStruct((n_rows, d), table.dtype),
        mesh=plsc.VectorSubcoreMesh(core_axis_name="c", subcore_axis_name="s"),
        scratch_types=[
            pltpu.VMEM((GCH,), jnp.int32),
            pltpu.VMEM((GCH, d), table.dtype),
            pltpu.SemaphoreType.DMA,
        ],
    )
    return k(table, idx)


def _expert_mlp_kernel(be_ref, xs_ref, gw_ref, uw_ref, dw_ref, y_ref):
    x = xs_ref[...].astype(jnp.bfloat16)
    g = lax.dot_general(x, gw_ref[0], (((1,), (1,)), ((), ())),
                        preferred_element_type=jnp.float32)
    u = lax.dot_general(x, uw_ref[0], (((1,), (1,)), ((), ())),
                        preferred_element_type=jnp.float32)
    h = (g * lax.logistic(g) * u).astype(jnp.bfloat16)
    y_ref[...] = lax.dot_general(h, dw_ref[0], (((1,), (1,)), ((), ())),
                                 preferred_element_type=jnp.float32)


def _expert_mlp(be, x_sorted, egw, euw, edw):
    p_rows, d = x_sorted.shape
    ff = egw.shape[1]
    nb = p_rows // RB
    grid_spec = pltpu.PrefetchScalarGridSpec(
        num_scalar_prefetch=1,
        grid=(nb,),
        in_specs=[
            pl.BlockSpec((RB, d), lambda i, be: (i, 0)),
            pl.BlockSpec((1, ff, d), lambda i, be: (be[i], 0, 0)),
            pl.BlockSpec((1, ff, d), lambda i, be: (be[i], 0, 0)),
            pl.BlockSpec((1, d, ff), lambda i, be: (be[i], 0, 0)),
        ],
        out_specs=pl.BlockSpec((RB, d), lambda i, be: (i, 0)),
    )
    return pl.pallas_call(
        _expert_mlp_kernel,
        grid_spec=grid_spec,
        out_shape=jax.ShapeDtypeStruct((p_rows, d), jnp.float32),
        compiler_params=pltpu.CompilerParams(vmem_limit_bytes=128 * 1024 * 1024),
    )(be, x_sorted, egw, euw, edw)


def _shared_combine_kernel(hs_ref, y0_ref, y1_ref, w_ref, sgate_ref,
                           sg_ref, su_ref, sd_ref, out_ref):
    x16 = hs_ref[...].astype(jnp.bfloat16)
    g = lax.dot_general(x16, sg_ref[...], (((1,), (1,)), ((), ())),
                        preferred_element_type=jnp.float32)
    u = lax.dot_general(x16, su_ref[...], (((1,), (1,)), ((), ())),
                        preferred_element_type=jnp.float32)
    h = (g * lax.logistic(g) * u).astype(jnp.bfloat16)
    s = lax.dot_general(h, sd_ref[...], (((1,), (1,)), ((), ())),
                        preferred_element_type=jnp.float32)
    w = w_ref[...]
    out_ref[...] = (w[:, 0:1] * y0_ref[...] + w[:, 1:2] * y1_ref[...]
                    + sgate_ref[...] * s)


def _shared_combine(hs, yg, w01, sgate, sgw, suw, sdw):
    T, d = hs.shape
    ff = sgw.shape[0]
    nch = T // CHUNK
    return pl.pallas_call(
        _shared_combine_kernel,
        grid=(nch,),
        in_specs=[
            pl.BlockSpec((CHUNK, d), lambda i: (i, 0)),
            pl.BlockSpec((CHUNK, d), lambda i: (i, 0)),
            pl.BlockSpec((CHUNK, d), lambda i: (i + nch, 0)),
            pl.BlockSpec((CHUNK, KTOP), lambda i: (i, 0)),
            pl.BlockSpec((CHUNK, 1), lambda i: (i, 0)),
            pl.BlockSpec((ff, d), lambda i: (0, 0)),
            pl.BlockSpec((ff, d), lambda i: (0, 0)),
            pl.BlockSpec((d, ff), lambda i: (0, 0)),
        ],
        out_specs=pl.BlockSpec((CHUNK, d), lambda i: (i, 0)),
        out_shape=jax.ShapeDtypeStruct((T, d), jnp.float32),
        compiler_params=pltpu.CompilerParams(vmem_limit_bytes=128 * 1024 * 1024),
    )(hs, yg, yg, w01, sgate, sgw, suw, sdw)


def kernel(hidden_states, gate_w, expert_gate_w, expert_up_w, expert_down_w,
           shared_gate_w, shared_up_w, shared_down_w, shared_expert_gate_w):
    B, S, Dm = hidden_states.shape
    hs = hidden_states.reshape(-1, Dm)
    T = hs.shape[0]
    nb = (KTOP * T + E * RB) // RB
    p_rows = nb * RB

    logits, w01, pos0, pos1, be, sgate = _router_plan(
        hs, gate_w, shared_expert_gate_w, nb)

    pos_flat = jnp.concatenate([pos0[:, 0], pos1[:, 0]])

    x_sorted = _sc_dispatch(hs, pos_flat, p_rows)
    y_sorted = _expert_mlp(be.reshape(nb), x_sorted,
                           expert_gate_w.astype(jnp.bfloat16),
                           expert_up_w.astype(jnp.bfloat16),
                           expert_down_w.astype(jnp.bfloat16))
    yg = _sc_gather_rows(y_sorted, pos_flat, KTOP * T)
    final = _shared_combine(hs, yg, w01, sgate,
                            shared_gate_w.astype(jnp.bfloat16),
                            shared_up_w.astype(jnp.bfloat16),
                            shared_down_w.astype(jnp.bfloat16))
    return final.reshape(B, S, Dm), logits
